# baseline (speedup 1.0000x reference)
import functools

import jax
import jax.numpy as jnp
from jax import lax
from jax.experimental import pallas as pl
from jax.experimental.pallas import tpu as pltpu
from jax.experimental.pallas import tpu_sc as plsc

D_MODEL = 64
MAX_REL = 128
SEQ_LEN = 2048
N_TABLE = 2 * MAX_REL + 1

NUM_CORES = 2
NUM_SUBCORES = 16
NW = NUM_CORES * NUM_SUBCORES
ROWS_PER_W = SEQ_LEN // NW
COL_HALF = SEQ_LEN // 2
WIN = COL_HALF + ROWS_PER_W
IDX_CHUNK = 128
N_CHUNKS = (WIN + IDX_CHUNK - 1) // IDX_CHUNK
WIN_PAD = N_CHUNKS * IDX_CHUNK
LANES = 16
WRITE_BATCH = 8


def _sc_body(table_hbm, out_hbm, win_ref, idx_ref, sem_g, sem_w):
    c = lax.axis_index("c")
    s = lax.axis_index("s")
    wid = c * NUM_SUBCORES + s
    row0 = wid * ROWS_PER_W

    for h in range(2):
        b0 = h * COL_HALF - row0 + (MAX_REL - ROWS_PER_W + 1)

        def gen_idx(t, carry):
            vals = b0 + t * LANES + lax.iota(jnp.int32, LANES)
            idx_ref[pl.ds(t * LANES, LANES)] = jnp.clip(vals, 0, N_TABLE - 1)
            return carry

        lax.fori_loop(0, WIN_PAD // LANES, gen_idx, 0)

        gathers = [
            pltpu.async_copy(
                table_hbm.at[idx_ref.at[pl.ds(k * IDX_CHUNK, IDX_CHUNK)]],
                win_ref.at[pl.ds(k * IDX_CHUNK, IDX_CHUNK)],
                sem_g,
            )
            for k in range(N_CHUNKS)
        ]
        for g in gathers:
            g.wait()

        writes = [
            pltpu.async_copy(
                win_ref.at[pl.ds(ROWS_PER_W - 1 - r, COL_HALF)],
                out_hbm.at[row0 + r, pl.ds(h * COL_HALF, COL_HALF)],
                sem_w,
            )
            for r in range(ROWS_PER_W)
        ]
        for hd in writes:
            hd.wait()


def kernel(seq_len, table):
    del seq_len
    mesh = plsc.VectorSubcoreMesh(
        core_axis_name="c", subcore_axis_name="s", num_cores=NUM_CORES
    )
    run = pl.kernel(
        _sc_body,
        out_type=jax.ShapeDtypeStruct((SEQ_LEN, SEQ_LEN, D_MODEL), jnp.float32),
        mesh=mesh,
        scratch_types=[
            pltpu.VMEM((WIN_PAD, D_MODEL), jnp.float32),
            pltpu.VMEM((WIN_PAD,), jnp.int32),
            pltpu.SemaphoreType.DMA,
            pltpu.SemaphoreType.DMA,
        ],
        compiler_params=pltpu.CompilerParams(use_tc_tiling_on_sc=False),
    )
    return run(table)


# speedup vs baseline: 7.7656x; 7.7656x over previous
import jax
import jax.numpy as jnp
from jax import lax
from jax.experimental import pallas as pl
from jax.experimental.pallas import tpu as pltpu
from jax.experimental.pallas import tpu_sc as plsc

D_MODEL = 64
MAX_REL = 128
SEQ_LEN = 2048
N_TABLE = 2 * MAX_REL + 1

NUM_CORES = 2
NUM_SUBCORES = 16
NW = NUM_CORES * NUM_SUBCORES
ROWS_PER_W = SEQ_LEN // NW
LPAD = SEQ_LEN - MAX_REL - 1
PTX_COLS = 4112
PT8_COLS = 4104

SUB = 8
LANE = 128
KD = D_MODEL // SUB
TJ = SEQ_LEN // LANE
NQ = 4
TQ = TJ // NQ
COL_Q = SEQ_LEN // NQ
BLK_COLS = LANE + ROWS_PER_W - SUB
NP = 8


def _sc_body(pt8_hbm, out_hbm, win_ref, sem_g, sem_w):
    c = lax.axis_index("c")
    s = lax.axis_index("s")
    wid = c * NUM_SUBCORES + s
    row0 = wid * ROWS_PER_W
    a = SEQ_LEN - ROWS_PER_W - row0

    n_stages = NQ * NP

    def issue_loads(s, b):
        h = lax.div(s, NP)
        p = lax.rem(s, NP)
        for t in range(TQ):
            pltpu.async_copy(
                pt8_hbm.at[p, :, pl.ds(a + h * COL_Q + t * LANE, BLK_COLS)],
                win_ref.at[b, t],
                sem_g,
            )

    def wait_loads():
        for t in range(TQ):
            pltpu.make_async_copy(
                pt8_hbm.at[0, :, pl.ds(0, BLK_COLS)],
                win_ref.at[0, t],
                sem_g,
            ).wait()

    def fire_writes(s, b):
        h = lax.div(s, NP)
        p = lax.rem(s, NP)

        def wq(q, carry):
            r = (NP - 1) - p + NP * q
            o = pl.multiple_of((ROWS_PER_W - NP) - NP * q, NP)
            for k in range(KD):
                pltpu.async_copy(
                    win_ref.at[b, :, pl.ds(k * SUB, SUB), pl.ds(o, LANE)],
                    out_hbm.at[row0 + r, k, pl.ds(h * TQ, TQ)],
                    sem_w,
                )
            return carry

        lax.fori_loop(0, ROWS_PER_W // NP, wq, 0)

    def drain_writes():
        def dq(q, carry):
            for k in range(KD):
                pltpu.make_async_copy(
                    win_ref.at[0, :, pl.ds(0, SUB), pl.ds(0, LANE)],
                    out_hbm.at[0, 0, pl.ds(0, TQ)],
                    sem_w,
                ).wait()
            return carry

        lax.fori_loop(0, ROWS_PER_W // NP, dq, 0)

    issue_loads(0, 0)

    def stage(s, carry):
        b = lax.rem(s, 2)
        @pl.when(s >= 1)
        def _():
            drain_writes()

        @pl.when(s <= n_stages - 2)
        def _():
            issue_loads(s + 1, 1 - b)

        wait_loads()
        fire_writes(s, b)
        return carry

    lax.fori_loop(0, n_stages, stage, 0)
    drain_writes()


def kernel(seq_len, table):
    del seq_len
    tt = jnp.transpose(table, (1, 0))
    ptx = jnp.concatenate(
        [
            jnp.broadcast_to(tt[:, :1], (D_MODEL, LPAD)),
            tt,
            jnp.broadcast_to(
                tt[:, N_TABLE - 1 :], (D_MODEL, PTX_COLS - LPAD - N_TABLE)
            ),
        ],
        axis=1,
    )
    pt8 = jnp.stack(
        [lax.slice(ptx, (0, p), (D_MODEL, p + PT8_COLS)) for p in range(NP)]
    )

    mesh = plsc.VectorSubcoreMesh(
        core_axis_name="c", subcore_axis_name="s", num_cores=NUM_CORES
    )
    run = pl.kernel(
        _sc_body,
        out_type=jax.ShapeDtypeStruct((SEQ_LEN, KD, TJ, SUB, LANE), jnp.float32),
        mesh=mesh,
        scratch_types=[
            pltpu.VMEM((2, TQ, D_MODEL, BLK_COLS), jnp.float32),
            pltpu.SemaphoreType.DMA,
            pltpu.SemaphoreType.DMA,
        ],
        compiler_params=pltpu.CompilerParams(use_tc_tiling_on_sc=False),
    )
    phys = run(pt8)
    out = jnp.transpose(phys, (0, 2, 4, 1, 3)).reshape(SEQ_LEN, SEQ_LEN, D_MODEL)
    return out


# speedup vs baseline: 7.7801x; 1.0019x over previous
import jax
import jax.numpy as jnp
from jax import lax
from jax.experimental import pallas as pl
from jax.experimental.pallas import tpu as pltpu
from jax.experimental.pallas import tpu_sc as plsc

D_MODEL = 64
MAX_REL = 128
SEQ_LEN = 2048
N_TABLE = 2 * MAX_REL + 1

NUM_CORES = 2
NUM_SUBCORES = 16
NW = NUM_CORES * NUM_SUBCORES
ROWS_PER_W = SEQ_LEN // NW
LPAD = SEQ_LEN - MAX_REL - 1
PTX_COLS = 4112
PT8_COLS = 4104

SUB = 8
LANE = 128
KD = D_MODEL // SUB
TJ = SEQ_LEN // LANE
NQ = 4
TQ = TJ // NQ
COL_Q = SEQ_LEN // NQ
BLK_COLS = LANE + ROWS_PER_W - SUB
NP = 8


def _sc_body(pt8_hbm, out_hbm, win_ref, sem_g, sem_w):
    c = lax.axis_index("c")
    s = lax.axis_index("s")
    wid = c * NUM_SUBCORES + s
    row0 = wid * ROWS_PER_W
    a = SEQ_LEN - ROWS_PER_W - row0

    n_stages = NQ * NP

    def issue_loads(s, b):
        h = lax.div(s, NP)
        p = lax.rem(s, NP)
        for t in range(TQ):
            pltpu.async_copy(
                pt8_hbm.at[p, :, :, pl.ds(a + h * COL_Q + t * LANE, BLK_COLS)],
                win_ref.at[b, :, t],
                sem_g,
            )

    def wait_loads():
        for t in range(TQ):
            pltpu.make_async_copy(
                pt8_hbm.at[0, :, :, pl.ds(0, BLK_COLS)],
                win_ref.at[0, :, t],
                sem_g,
            ).wait()

    def fire_writes(s, b):
        h = lax.div(s, NP)
        p = lax.rem(s, NP)

        def wq(q, carry):
            r = (NP - 1) - p + NP * q
            o = pl.multiple_of((ROWS_PER_W - NP) - NP * q, NP)
            pltpu.async_copy(
                win_ref.at[b, :, :, :, pl.ds(o, LANE)],
                out_hbm.at[row0 + r, :, pl.ds(h * TQ, TQ)],
                sem_w,
            )
            return carry

        lax.fori_loop(0, ROWS_PER_W // NP, wq, 0)

    def drain_writes():
        def dq(q, carry):
            pltpu.make_async_copy(
                win_ref.at[0, :, :, :, pl.ds(0, LANE)],
                out_hbm.at[0, :, pl.ds(0, TQ)],
                sem_w,
            ).wait()
            return carry

        lax.fori_loop(0, ROWS_PER_W // NP, dq, 0)

    issue_loads(0, 0)

    def stage(s, carry):
        b = lax.rem(s, 2)
        @pl.when(s >= 1)
        def _():
            drain_writes()

        @pl.when(s <= n_stages - 2)
        def _():
            issue_loads(s + 1, 1 - b)

        wait_loads()
        fire_writes(s, b)
        return carry

    lax.fori_loop(0, n_stages, stage, 0)
    drain_writes()


def kernel(seq_len, table):
    del seq_len
    tt = jnp.transpose(table, (1, 0))
    ptx = jnp.concatenate(
        [
            jnp.broadcast_to(tt[:, :1], (D_MODEL, LPAD)),
            tt,
            jnp.broadcast_to(
                tt[:, N_TABLE - 1 :], (D_MODEL, PTX_COLS - LPAD - N_TABLE)
            ),
        ],
        axis=1,
    )
    pt8 = jnp.stack(
        [lax.slice(ptx, (0, p), (D_MODEL, p + PT8_COLS)) for p in range(NP)]
    ).reshape(NP, KD, SUB, PT8_COLS)

    mesh = plsc.VectorSubcoreMesh(
        core_axis_name="c", subcore_axis_name="s", num_cores=NUM_CORES
    )
    run = pl.kernel(
        _sc_body,
        out_type=jax.ShapeDtypeStruct((SEQ_LEN, KD, TJ, SUB, LANE), jnp.float32),
        mesh=mesh,
        scratch_types=[
            pltpu.VMEM((2, KD, TQ, SUB, BLK_COLS), jnp.float32),
            pltpu.SemaphoreType.DMA,
            pltpu.SemaphoreType.DMA,
        ],
        compiler_params=pltpu.CompilerParams(use_tc_tiling_on_sc=False),
    )
    phys = run(pt8)
    out = jnp.transpose(phys, (0, 2, 4, 1, 3)).reshape(SEQ_LEN, SEQ_LEN, D_MODEL)
    return out
